# baseline (speedup 1.0000x reference)
import functools
import jax
import jax.numpy as jnp
from jax import lax
from jax.experimental import pallas as pl
from jax.experimental.pallas import tpu as pltpu
from jax.experimental.pallas import tpu_sc as plsc

N = 4096
D = 512
TOPK = 32
NEG = float("-inf")
NW = 32
RPW = N // NW
PAD = 64


def _thresh_body(w_ref, t_ref, rinv_ref):
    w = w_ref[...]
    B = w.shape[0]
    xs = [w[:, 128 * j : 128 * (j + 1)] for j in range(32)]
    k = 2
    while k <= 32:
        j = k // 2
        while j >= 1:
            for i in range(32):
                l = i ^ j
                if l > i:
                    a, b = xs[i], xs[l]
                    hi = jnp.maximum(a, b)
                    lo = jnp.minimum(a, b)
                    if (i & k) == 0:
                        xs[i], xs[l] = lo, hi
                    else:
                        xs[i], xs[l] = hi, lo
            j //= 2
        k *= 2
    rest = xs[24:31][::-1]

    def pop(_, state):
        heads, depth, _ = state
        m = jnp.max(heads, axis=1, keepdims=True)
        hit = heads == m
        depth = depth + hit.astype(jnp.int32)
        nxt = jnp.full_like(heads, NEG)
        for d_i in range(6, -1, -1):
            nxt = jnp.where(depth == d_i + 1, rest[d_i], nxt)
        heads = jnp.where(hit, nxt, heads)
        return (heads, depth, m)

    _, _, t = lax.fori_loop(
        0,
        TOPK,
        pop,
        (xs[31], jnp.zeros((B, 128), jnp.int32), jnp.zeros((B, 1), jnp.float32)),
    )
    wsp = jnp.where(w >= t, w, 0.0)
    rs = jnp.sum(wsp, axis=1, keepdims=True) + 1e-8
    t_ref[...] = jnp.broadcast_to(t, (B, 16))
    rinv_ref[...] = jnp.broadcast_to(1.0 / rs, (B, 16))


def _vproj_body(h_ref, Wv_ref, bv_ref, out_ref):
    out_ref[...] = (
        jnp.dot(h_ref[...], Wv_ref[...], preferred_element_type=jnp.float32)
        + bv_ref[...]
    )


def _gate_body(h_ref, msg_ref, Wo_ref, bo_ref, Wg_ref, bg_ref, out_ref):
    h = h_ref[...]
    msg = msg_ref[...]
    out = jnp.dot(msg, Wo_ref[...], preferred_element_type=jnp.float32) + bo_ref[...]
    alpha = jax.nn.sigmoid(
        jnp.dot(h, Wg_ref[...], preferred_element_type=jnp.float32) + bg_ref[...]
    )
    out_ref[...] = alpha * h + (1.0 - alpha) * out


_SC_MESH = plsc.VectorSubcoreMesh(core_axis_name="c", subcore_axis_name="s")


def _sc_wid():
    return lax.axis_index("s") * 2 + lax.axis_index("c")


@functools.partial(
    pl.kernel,
    out_type=[
        jax.ShapeDtypeStruct((N * PAD,), jnp.int32),
        jax.ShapeDtypeStruct((N * PAD,), jnp.float32),
    ],
    mesh=_SC_MESH,
    compiler_params=pltpu.CompilerParams(needs_layout_passes=False),
    scratch_types=[
        pltpu.VMEM((N,), jnp.float32),
        pltpu.VMEM((N,), jnp.float32),
        pltpu.VMEM((RPW * 16,), jnp.float32),
        pltpu.VMEM((RPW * 16,), jnp.float32),
        pltpu.VMEM((RPW * PAD,), jnp.int32),
        pltpu.VMEM((RPW * PAD,), jnp.float32),
        pltpu.SemaphoreType.DMA,
        pltpu.SemaphoreType.DMA,
    ],
)
def _sc_compact(w_hbm, t_hbm, rinv_hbm, idx_hbm, wn_hbm,
                buf0, buf1, tbuf, rbuf, sidx, swn, sem0, sem1):
    wid = _sc_wid()
    base = wid * RPW
    pltpu.sync_copy(t_hbm.at[pl.ds(base * 16, RPW * 16)], tbuf)
    pltpu.sync_copy(rinv_hbm.at[pl.ds(base * 16, RPW * 16)], rbuf)

    def clear(j, _):
        sidx[pl.ds(j * 16, 16)] = jnp.zeros((16,), jnp.int32)
        swn[pl.ds(j * 16, 16)] = jnp.zeros((16,), jnp.float32)
        return 0

    lax.fori_loop(0, RPW * PAD // 16, clear, 0)

    def scan_row(i, buf):
        tv = tbuf[pl.ds(i * 16, 16)]
        rv = rbuf[pl.ds(i * 16, 16)]
        ones = jnp.full((16,), 1, jnp.int32)
        zeros = jnp.full((16,), 0, jnp.int32)
        cap = jnp.full((16,), PAD - 1, jnp.int32)

        def chunk(c, cntv):
            v = buf[pl.ds(c * 16, 16)]
            m = v >= tv
            ids = lax.iota(jnp.int32, 16) + c * 16
            pc = plsc.cumsum(jnp.where(m, ones, zeros))
            pos = jnp.minimum(cntv + pc - 1, cap) + i * PAD
            plsc.store_scatter(sidx, [pos], ids, mask=m)
            plsc.store_scatter(swn, [pos], v * rv, mask=m)
            return cntv + plsc.all_reduce_population_count(m)

        lax.fori_loop(0, N // 16, chunk, jnp.full((16,), 0, jnp.int32))

    def issue(r, buf, sem):
        pltpu.async_copy(w_hbm.at[pl.ds((base + r) * N, N)], buf, sem)

    def drain(buf, sem):
        pltpu.make_async_copy(w_hbm.at[pl.ds(base * N, N)], buf, sem).wait()

    issue(0, buf0, sem0)

    def pair(p, _):
        r0 = 2 * p
        r1 = r0 + 1
        issue(r1, buf1, sem1)
        drain(buf0, sem0)
        scan_row(r0, buf0)

        @pl.when(r1 + 1 < RPW)
        def _():
            issue(r1 + 1, buf0, sem0)

        drain(buf1, sem1)
        scan_row(r1, buf1)
        return 0

    lax.fori_loop(0, RPW // 2, pair, 0)
    pltpu.sync_copy(sidx, idx_hbm.at[pl.ds(base * PAD, RPW * PAD)])
    pltpu.sync_copy(swn, wn_hbm.at[pl.ds(base * PAD, RPW * PAD)])


@functools.partial(
    pl.kernel,
    out_type=jax.ShapeDtypeStruct((N * D,), jnp.float32),
    mesh=_SC_MESH,
    compiler_params=pltpu.CompilerParams(needs_layout_passes=False),
    scratch_types=[
        pltpu.VMEM((TOPK, D), jnp.float32),
        pltpu.VMEM((TOPK, D), jnp.float32),
        pltpu.VMEM((RPW * PAD,), jnp.int32),
        pltpu.VMEM((RPW * PAD,), jnp.float32),
        pltpu.VMEM((RPW * D,), jnp.float32),
        pltpu.SemaphoreType.DMA,
        pltpu.SemaphoreType.DMA,
    ],
)
def _sc_aggregate(v_hbm, idx_hbm, wn_hbm, out_hbm,
                  rows0, rows1, idxb, wnb, outb, sem0, sem1):
    wid = _sc_wid()
    base = wid * RPW
    pltpu.sync_copy(idx_hbm.at[pl.ds(base * PAD, RPW * PAD)], idxb)
    pltpu.sync_copy(wn_hbm.at[pl.ds(base * PAD, RPW * PAD)], wnb)

    def gather(i, rows, sem):
        pltpu.async_copy(v_hbm.at[idxb.at[pl.ds(i * PAD, TOPK)]], rows, sem)

    def drain(rows, sem):
        pltpu.make_async_copy(
            v_hbm.at[idxb.at[pl.ds(0, TOPK)]], rows, sem
        ).wait()

    def row_body(i, rows):
        def nbr(k, acc):
            ws = plsc.load_gather(wnb, [jnp.full((16,), i * PAD + k, jnp.int32)])
            return [
                acc[v] + ws * rows[k, pl.ds(v * 16, 16)]
                for v in range(D // 16)
            ]

        acc = lax.fori_loop(
            0, TOPK, nbr, [jnp.zeros((16,), jnp.float32)] * (D // 16)
        )
        for v in range(D // 16):
            outb[pl.ds(i * D + v * 16, 16)] = acc[v]

    gather(0, rows0, sem0)

    def pair(p, _):
        r0 = 2 * p
        r1 = r0 + 1
        gather(r1, rows1, sem1)
        drain(rows0, sem0)
        row_body(r0, rows0)

        @pl.when(r1 + 1 < RPW)
        def _():
            gather(r1 + 1, rows0, sem0)

        drain(rows1, sem1)
        row_body(r1, rows1)
        return 0

    lax.fori_loop(0, RPW // 2, pair, 0)
    pltpu.sync_copy(outb, out_hbm.at[pl.ds(base * D, RPW * D)])


@jax.jit
def kernel(h, w, Wv0, bv0, Wo0, bo0, Wv1, bv1, Wo1, bo1, Wg, bg):
    BT = 256
    BR = 512

    t16, rinv16 = pl.pallas_call(
        _thresh_body,
        grid=(N // BT,),
        in_specs=[pl.BlockSpec((BT, N), lambda i: (i, 0))],
        out_specs=[
            pl.BlockSpec((BT, 16), lambda i: (i, 0)),
            pl.BlockSpec((BT, 16), lambda i: (i, 0)),
        ],
        out_shape=[
            jax.ShapeDtypeStruct((N, 16), jnp.float32),
            jax.ShapeDtypeStruct((N, 16), jnp.float32),
        ],
    )(w)

    idx64, wn64 = _sc_compact(
        w.reshape(-1), t16.reshape(-1), rinv16.reshape(-1)
    )

    vproj = pl.pallas_call(
        _vproj_body,
        grid=(N // BR,),
        in_specs=[
            pl.BlockSpec((BR, D), lambda i: (i, 0)),
            pl.BlockSpec((D, D), lambda i: (0, 0)),
            pl.BlockSpec((1, D), lambda i: (0, 0)),
        ],
        out_specs=pl.BlockSpec((BR, D), lambda i: (i, 0)),
        out_shape=jax.ShapeDtypeStruct((N, D), jnp.float32),
    )

    gate = pl.pallas_call(
        _gate_body,
        grid=(N // BR,),
        in_specs=[
            pl.BlockSpec((BR, D), lambda i: (i, 0)),
            pl.BlockSpec((BR, D), lambda i: (i, 0)),
            pl.BlockSpec((D, D), lambda i: (0, 0)),
            pl.BlockSpec((1, D), lambda i: (0, 0)),
            pl.BlockSpec((D, 1), lambda i: (0, 0)),
            pl.BlockSpec((1, 1), lambda i: (0, 0)),
        ],
        out_specs=pl.BlockSpec((BR, D), lambda i: (i, 0)),
        out_shape=jax.ShapeDtypeStruct((N, D), jnp.float32),
    )

    bg2 = bg.reshape(1, 1)
    for (Wv, bv, Wo, bo) in ((Wv0, bv0, Wo0, bo0), (Wv1, bv1, Wo1, bo1)):
        V = vproj(h, Wv, bv.reshape(1, D))
        msg = _sc_aggregate(V, idx64, wn64).reshape(N, D)
        h = gate(h, msg, Wo, bo.reshape(1, D), Wg, bg2)
    return h


# speedup vs baseline: 1.3308x; 1.3308x over previous
import functools
import jax
import jax.numpy as jnp
from jax import lax
from jax.experimental import pallas as pl
from jax.experimental.pallas import tpu as pltpu
from jax.experimental.pallas import tpu_sc as plsc

N = 4096
D = 512
TOPK = 32
NEG = float("-inf")
NW = 32
RPW = N // NW
PAD = 64


def _thresh_body(w_ref, t_ref, rinv_ref, gmax_ref):
    w = w_ref[...]
    B = w.shape[0]
    xs = [w[:, 128 * j : 128 * (j + 1)] for j in range(32)]
    k = 2
    while k <= 32:
        j = k // 2
        while j >= 1:
            for i in range(32):
                l = i ^ j
                if l > i:
                    a, b = xs[i], xs[l]
                    hi = jnp.maximum(a, b)
                    lo = jnp.minimum(a, b)
                    if (i & k) == 0:
                        xs[i], xs[l] = lo, hi
                    else:
                        xs[i], xs[l] = hi, lo
            j //= 2
        k *= 2
    rest = xs[24:31][::-1]

    def pop(_, state):
        heads, depth, _ = state
        m = jnp.max(heads, axis=1, keepdims=True)
        hit = heads == m
        depth = depth + hit.astype(jnp.int32)
        nxt = jnp.full_like(heads, NEG)
        for d_i in range(6, -1, -1):
            nxt = jnp.where(depth == d_i + 1, rest[d_i], nxt)
        heads = jnp.where(hit, nxt, heads)
        return (heads, depth, m)

    _, _, t = lax.fori_loop(
        0,
        TOPK,
        pop,
        (xs[31], jnp.zeros((B, 128), jnp.int32), jnp.zeros((B, 1), jnp.float32)),
    )
    wsp = jnp.where(w >= t, w, 0.0)
    rs = jnp.sum(wsp, axis=1, keepdims=True) + 1e-8
    t_ref[...] = jnp.broadcast_to(t, (B, 16))
    rinv_ref[...] = jnp.broadcast_to(1.0 / rs, (B, 16))
    gmax_ref[...] = xs[31]


def _vproj_body(h_ref, Wv_ref, bv_ref, out_ref):
    out_ref[...] = (
        jnp.dot(h_ref[...], Wv_ref[...], preferred_element_type=jnp.float32)
        + bv_ref[...]
    )


def _gate_body(h_ref, msg_ref, Wo_ref, bo_ref, Wg_ref, bg_ref, out_ref):
    h = h_ref[...]
    msg = msg_ref[...]
    out = jnp.dot(msg, Wo_ref[...], preferred_element_type=jnp.float32) + bo_ref[...]
    alpha = jax.nn.sigmoid(
        jnp.dot(h, Wg_ref[...], preferred_element_type=jnp.float32) + bg_ref[...]
    )
    out_ref[...] = alpha * h + (1.0 - alpha) * out


_SC_MESH = plsc.VectorSubcoreMesh(core_axis_name="c", subcore_axis_name="s")


def _sc_wid():
    return lax.axis_index("s") * 2 + lax.axis_index("c")


@functools.partial(
    pl.kernel,
    out_type=[
        jax.ShapeDtypeStruct((N * PAD,), jnp.int32),
        jax.ShapeDtypeStruct((N * PAD,), jnp.float32),
    ],
    mesh=_SC_MESH,
    compiler_params=pltpu.CompilerParams(needs_layout_passes=False),
    scratch_types=[
        pltpu.VMEM((N,), jnp.float32),
        pltpu.VMEM((N,), jnp.float32),
        pltpu.VMEM((RPW * 16,), jnp.float32),
        pltpu.VMEM((RPW * 16,), jnp.float32),
        pltpu.VMEM((RPW * 128,), jnp.float32),
        pltpu.VMEM((32,), jnp.int32),
        pltpu.VMEM((RPW * PAD,), jnp.int32),
        pltpu.VMEM((RPW * PAD,), jnp.float32),
        pltpu.SemaphoreType.DMA,
        pltpu.SemaphoreType.DMA,
    ],
)
def _sc_compact(w_hbm, t_hbm, rinv_hbm, gmax_hbm, idx_hbm, wn_hbm,
                buf0, buf1, tbuf, rbuf, gbuf, glist, sidx, swn, sem0, sem1):
    wid = _sc_wid()
    base = wid * RPW
    pltpu.sync_copy(t_hbm.at[pl.ds(base * 16, RPW * 16)], tbuf)
    pltpu.sync_copy(rinv_hbm.at[pl.ds(base * 16, RPW * 16)], rbuf)
    pltpu.sync_copy(gmax_hbm.at[pl.ds(base * 128, RPW * 128)], gbuf)

    iot = lax.iota(jnp.int32, 16)
    ones = jnp.full((16,), 1, jnp.int32)
    zeros = jnp.full((16,), 0, jnp.int32)

    def scan_row(i, buf):
        tv = tbuf[pl.ds(i * 16, 16)]
        rv = rbuf[pl.ds(i * 16, 16)]
        cg = zeros
        for j in range(8):
            gv = gbuf[pl.ds(i * 128 + j * 16, 16)]
            mg = gv >= tv
            pc = plsc.cumsum(jnp.where(mg, ones, zeros))
            pos = jnp.minimum(cg + pc - 1, jnp.full((16,), 31, jnp.int32))
            plsc.store_scatter(glist, [pos], iot + j * 16, mask=mg)
            cg = cg + plsc.all_reduce_population_count(mg)
        cnt = zeros
        for b in range(2):
            gids = glist[pl.ds(b * 16, 16)]
            lanem = (iot + b * 16) < cg
            for k in range(32):
                addr = gids + k * 128
                vals = plsc.load_gather(buf, [addr], mask=lanem)
                m = (vals >= tv) & lanem
                pc = plsc.cumsum(jnp.where(m, ones, zeros))
                pos = (
                    jnp.minimum(cnt + pc - 1, jnp.full((16,), PAD - 1, jnp.int32))
                    + i * PAD
                )
                plsc.store_scatter(sidx, [pos], addr, mask=m)
                plsc.store_scatter(swn, [pos], vals * rv, mask=m)
                cnt = cnt + plsc.all_reduce_population_count(m)

    def issue(r, buf, sem):
        pltpu.async_copy(w_hbm.at[base + r], buf, sem)

    def drain(buf, sem):
        pltpu.make_async_copy(w_hbm.at[base], buf, sem).wait()

    issue(0, buf0, sem0)

    def pair(p, _):
        r0 = 2 * p
        r1 = r0 + 1
        issue(r1, buf1, sem1)
        drain(buf0, sem0)
        scan_row(r0, buf0)

        @pl.when(r1 + 1 < RPW)
        def _():
            issue(r1 + 1, buf0, sem0)

        drain(buf1, sem1)
        scan_row(r1, buf1)
        return 0

    lax.fori_loop(0, RPW // 2, pair, 0)
    pltpu.sync_copy(sidx, idx_hbm.at[pl.ds(base * PAD, RPW * PAD)])
    pltpu.sync_copy(swn, wn_hbm.at[pl.ds(base * PAD, RPW * PAD)])


@functools.partial(
    pl.kernel,
    out_type=jax.ShapeDtypeStruct((N * D,), jnp.float32),
    mesh=_SC_MESH,
    compiler_params=pltpu.CompilerParams(needs_layout_passes=False),
    scratch_types=[
        pltpu.VMEM((TOPK, D), jnp.float32),
        pltpu.VMEM((TOPK, D), jnp.float32),
        pltpu.VMEM((RPW * PAD,), jnp.int32),
        pltpu.VMEM((RPW * PAD,), jnp.float32),
        pltpu.VMEM((RPW * D,), jnp.float32),
        pltpu.SemaphoreType.DMA,
        pltpu.SemaphoreType.DMA,
    ],
)
def _sc_aggregate(v_hbm, idx_hbm, wn_hbm, out_hbm,
                  rows0, rows1, idxb, wnb, outb, sem0, sem1):
    wid = _sc_wid()
    base = wid * RPW
    pltpu.sync_copy(idx_hbm.at[pl.ds(base * PAD, RPW * PAD)], idxb)
    pltpu.sync_copy(wn_hbm.at[pl.ds(base * PAD, RPW * PAD)], wnb)

    def gather(i, rows, sem):
        pltpu.async_copy(v_hbm.at[idxb.at[pl.ds(i * PAD, TOPK)]], rows, sem)

    def drain(rows, sem):
        pltpu.make_async_copy(
            v_hbm.at[idxb.at[pl.ds(0, TOPK)]], rows, sem
        ).wait()

    def row_body(i, rows):
        def nbr(k, acc):
            ws = plsc.load_gather(wnb, [jnp.full((16,), i * PAD + k, jnp.int32)])
            return [
                acc[v] + ws * rows[k, pl.ds(v * 16, 16)]
                for v in range(D // 16)
            ]

        acc = lax.fori_loop(
            0, TOPK, nbr, [jnp.zeros((16,), jnp.float32)] * (D // 16)
        )
        for v in range(D // 16):
            outb[pl.ds(i * D + v * 16, 16)] = acc[v]

    gather(0, rows0, sem0)

    def pair(p, _):
        r0 = 2 * p
        r1 = r0 + 1
        gather(r1, rows1, sem1)
        drain(rows0, sem0)
        row_body(r0, rows0)

        @pl.when(r1 + 1 < RPW)
        def _():
            gather(r1 + 1, rows0, sem0)

        drain(rows1, sem1)
        row_body(r1, rows1)
        return 0

    lax.fori_loop(0, RPW // 2, pair, 0)
    pltpu.sync_copy(outb, out_hbm.at[pl.ds(base * D, RPW * D)])


@jax.jit
def kernel(h, w, Wv0, bv0, Wo0, bo0, Wv1, bv1, Wo1, bo1, Wg, bg):
    BT = 256
    BR = 512

    t16, rinv16, gmax = pl.pallas_call(
        _thresh_body,
        grid=(N // BT,),
        in_specs=[pl.BlockSpec((BT, N), lambda i: (i, 0))],
        out_specs=[
            pl.BlockSpec((BT, 16), lambda i: (i, 0)),
            pl.BlockSpec((BT, 16), lambda i: (i, 0)),
            pl.BlockSpec((BT, 128), lambda i: (i, 0)),
        ],
        out_shape=[
            jax.ShapeDtypeStruct((N, 16), jnp.float32),
            jax.ShapeDtypeStruct((N, 16), jnp.float32),
            jax.ShapeDtypeStruct((N, 128), jnp.float32),
        ],
    )(w)

    idx64, wn64 = _sc_compact(
        w, t16.reshape(-1), rinv16.reshape(-1), gmax.reshape(-1)
    )

    vproj = pl.pallas_call(
        _vproj_body,
        grid=(N // BR,),
        in_specs=[
            pl.BlockSpec((BR, D), lambda i: (i, 0)),
            pl.BlockSpec((D, D), lambda i: (0, 0)),
            pl.BlockSpec((1, D), lambda i: (0, 0)),
        ],
        out_specs=pl.BlockSpec((BR, D), lambda i: (i, 0)),
        out_shape=jax.ShapeDtypeStruct((N, D), jnp.float32),
    )

    gate = pl.pallas_call(
        _gate_body,
        grid=(N // BR,),
        in_specs=[
            pl.BlockSpec((BR, D), lambda i: (i, 0)),
            pl.BlockSpec((BR, D), lambda i: (i, 0)),
            pl.BlockSpec((D, D), lambda i: (0, 0)),
            pl.BlockSpec((1, D), lambda i: (0, 0)),
            pl.BlockSpec((D, 1), lambda i: (0, 0)),
            pl.BlockSpec((1, 1), lambda i: (0, 0)),
        ],
        out_specs=pl.BlockSpec((BR, D), lambda i: (i, 0)),
        out_shape=jax.ShapeDtypeStruct((N, D), jnp.float32),
    )

    bg2 = bg.reshape(1, 1)
    for (Wv, bv, Wo, bo) in ((Wv0, bv0, Wo0, bo0), (Wv1, bv1, Wo1, bo1)):
        V = vproj(h, Wv, bv.reshape(1, D))
        msg = _sc_aggregate(V, idx64, wn64).reshape(N, D)
        h = gate(h, msg, Wo, bo.reshape(1, D), Wg, bg2)
    return h


# speedup vs baseline: 1.3470x; 1.0121x over previous
import functools
import jax
import jax.numpy as jnp
from jax import lax
from jax.experimental import pallas as pl
from jax.experimental.pallas import tpu as pltpu
from jax.experimental.pallas import tpu_sc as plsc

N = 4096
D = 512
TOPK = 32
NEG = float("-inf")
NW = 32
RPW = N // NW
PAD = 64


def _thresh_body(w_ref, t_ref, rinv_ref, gmax_ref):
    w = w_ref[...]
    B = w.shape[0]
    xs = [w[:, 128 * j : 128 * (j + 1)] for j in range(32)]
    k = 2
    while k <= 32:
        j = k // 2
        while j >= 1:
            for i in range(32):
                l = i ^ j
                if l > i:
                    a, b = xs[i], xs[l]
                    hi = jnp.maximum(a, b)
                    lo = jnp.minimum(a, b)
                    if (i & k) == 0:
                        xs[i], xs[l] = lo, hi
                    else:
                        xs[i], xs[l] = hi, lo
            j //= 2
        k *= 2
    rest = xs[24:31][::-1]

    def pop(_, state):
        heads, depth, _ = state
        m = jnp.max(heads, axis=1, keepdims=True)
        hit = heads == m
        depth = depth + hit.astype(jnp.int32)
        nxt = jnp.full_like(heads, NEG)
        for d_i in range(6, -1, -1):
            nxt = jnp.where(depth == d_i + 1, rest[d_i], nxt)
        heads = jnp.where(hit, nxt, heads)
        return (heads, depth, m)

    _, _, t = lax.fori_loop(
        0,
        TOPK,
        pop,
        (xs[31], jnp.zeros((B, 128), jnp.int32), jnp.zeros((B, 1), jnp.float32)),
    )
    wsp = jnp.where(w >= t, w, 0.0)
    rs = jnp.sum(wsp, axis=1, keepdims=True) + 1e-8
    t_ref[...] = jnp.broadcast_to(t, (B, 16))
    rinv_ref[...] = jnp.broadcast_to(1.0 / rs, (B, 16))
    gmax_ref[...] = xs[31]


def _vproj_body(h_ref, Wv_ref, bv_ref, out_ref):
    out_ref[...] = (
        jnp.dot(h_ref[...], Wv_ref[...], preferred_element_type=jnp.float32)
        + bv_ref[...]
    )


def _gate_body(h_ref, msg_ref, Wo_ref, bo_ref, Wg_ref, bg_ref, out_ref):
    h = h_ref[...]
    msg = msg_ref[...]
    out = jnp.dot(msg, Wo_ref[...], preferred_element_type=jnp.float32) + bo_ref[...]
    alpha = jax.nn.sigmoid(
        jnp.dot(h, Wg_ref[...], preferred_element_type=jnp.float32) + bg_ref[...]
    )
    out_ref[...] = alpha * h + (1.0 - alpha) * out


_SC_MESH = plsc.VectorSubcoreMesh(core_axis_name="c", subcore_axis_name="s")


def _sc_wid():
    return lax.axis_index("s") * 2 + lax.axis_index("c")


@functools.partial(
    pl.kernel,
    out_type=[
        jax.ShapeDtypeStruct((N * PAD,), jnp.int32),
        jax.ShapeDtypeStruct((N * PAD,), jnp.float32),
    ],
    mesh=_SC_MESH,
    compiler_params=pltpu.CompilerParams(needs_layout_passes=False),
    scratch_types=[
        pltpu.VMEM((N,), jnp.float32),
        pltpu.VMEM((N,), jnp.float32),
        pltpu.VMEM((N,), jnp.float32),
        pltpu.VMEM((N,), jnp.float32),
        pltpu.VMEM((RPW * 16,), jnp.float32),
        pltpu.VMEM((RPW * 16,), jnp.float32),
        pltpu.VMEM((RPW * 128,), jnp.float32),
        pltpu.VMEM((32,), jnp.int32),
        pltpu.VMEM((RPW * PAD,), jnp.int32),
        pltpu.VMEM((RPW * PAD,), jnp.float32),
        pltpu.SemaphoreType.DMA,
        pltpu.SemaphoreType.DMA,
        pltpu.SemaphoreType.DMA,
        pltpu.SemaphoreType.DMA,
    ],
)
def _sc_compact(w_hbm, t_hbm, rinv_hbm, gmax_hbm, idx_hbm, wn_hbm,
                buf0, buf1, buf2, buf3, tbuf, rbuf, gbuf, glist, sidx, swn,
                sem0, sem1, sem2, sem3):
    wid = _sc_wid()
    base = wid * RPW
    pltpu.sync_copy(t_hbm.at[pl.ds(base * 16, RPW * 16)], tbuf)
    pltpu.sync_copy(rinv_hbm.at[pl.ds(base * 16, RPW * 16)], rbuf)
    pltpu.sync_copy(gmax_hbm.at[pl.ds(base * 128, RPW * 128)], gbuf)

    iot = lax.iota(jnp.int32, 16)
    ones = jnp.full((16,), 1, jnp.int32)
    zeros = jnp.full((16,), 0, jnp.int32)

    def scan_row(i, buf):
        tv = tbuf[pl.ds(i * 16, 16)]
        rv = rbuf[pl.ds(i * 16, 16)]
        cg = zeros
        for j in range(8):
            gv = gbuf[pl.ds(i * 128 + j * 16, 16)]
            mg = gv >= tv
            pc = plsc.cumsum(jnp.where(mg, ones, zeros))
            pos = jnp.minimum(cg + pc - 1, jnp.full((16,), 31, jnp.int32))
            plsc.store_scatter(glist, [pos], iot + j * 16, mask=mg)
            cg = cg + plsc.all_reduce_population_count(mg)
        cnt = zeros
        for b in range(2):
            gids = glist[pl.ds(b * 16, 16)]
            lanem = (iot + b * 16) < cg
            for k in range(32):
                addr = gids + k * 128
                vals = plsc.load_gather(buf, [addr], mask=lanem)
                m = (vals >= tv) & lanem
                pc = plsc.cumsum(jnp.where(m, ones, zeros))
                pos = cnt + pc - 1 + i * PAD
                plsc.store_scatter(sidx, [pos], addr, mask=m)
                plsc.store_scatter(swn, [pos], vals * rv, mask=m)
                cnt = cnt + plsc.all_reduce_population_count(m)

    def issue(r, buf, sem):
        pltpu.async_copy(w_hbm.at[base + r], buf, sem)

    def drain(buf, sem):
        pltpu.make_async_copy(w_hbm.at[base], buf, sem).wait()

    bufs = (buf0, buf1, buf2, buf3)
    sems = (sem0, sem1, sem2, sem3)
    for j in range(3):
        issue(j, bufs[j], sems[j])

    def quad(q, _):
        r = 4 * q
        for j in range(4):
            nxt = r + j + 3

            @pl.when(nxt < RPW)
            def _():
                issue(nxt, bufs[(j + 3) % 4], sems[(j + 3) % 4])

            drain(bufs[j], sems[j])
            scan_row(r + j, bufs[j])
        return 0

    lax.fori_loop(0, RPW // 4, quad, 0)
    pltpu.sync_copy(sidx, idx_hbm.at[pl.ds(base * PAD, RPW * PAD)])
    pltpu.sync_copy(swn, wn_hbm.at[pl.ds(base * PAD, RPW * PAD)])


@functools.partial(
    pl.kernel,
    out_type=jax.ShapeDtypeStruct((N * D,), jnp.float32),
    mesh=_SC_MESH,
    compiler_params=pltpu.CompilerParams(needs_layout_passes=False),
    scratch_types=[
        pltpu.VMEM((TOPK, D), jnp.float32),
        pltpu.VMEM((TOPK, D), jnp.float32),
        pltpu.VMEM((RPW * PAD,), jnp.int32),
        pltpu.VMEM((RPW * PAD,), jnp.float32),
        pltpu.VMEM((RPW * D,), jnp.float32),
        pltpu.SemaphoreType.DMA,
        pltpu.SemaphoreType.DMA,
    ],
)
def _sc_aggregate(v_hbm, idx_hbm, wn_hbm, out_hbm,
                  rows0, rows1, idxb, wnb, outb, sem0, sem1):
    wid = _sc_wid()
    base = wid * RPW
    pltpu.sync_copy(idx_hbm.at[pl.ds(base * PAD, RPW * PAD)], idxb)
    pltpu.sync_copy(wn_hbm.at[pl.ds(base * PAD, RPW * PAD)], wnb)

    def gather(i, rows, sem):
        pltpu.async_copy(v_hbm.at[idxb.at[pl.ds(i * PAD, TOPK)]], rows, sem)

    def drain(rows, sem):
        pltpu.make_async_copy(
            v_hbm.at[idxb.at[pl.ds(0, TOPK)]], rows, sem
        ).wait()

    def row_body(i, rows):
        def nbr(k, acc):
            ws = plsc.load_gather(wnb, [jnp.full((16,), i * PAD + k, jnp.int32)])
            return [
                acc[v] + ws * rows[k, pl.ds(v * 16, 16)]
                for v in range(D // 16)
            ]

        acc = lax.fori_loop(
            0, TOPK, nbr, [jnp.zeros((16,), jnp.float32)] * (D // 16)
        )
        for v in range(D // 16):
            outb[pl.ds(i * D + v * 16, 16)] = acc[v]

    gather(0, rows0, sem0)

    def pair(p, _):
        r0 = 2 * p
        r1 = r0 + 1
        gather(r1, rows1, sem1)
        drain(rows0, sem0)
        row_body(r0, rows0)

        @pl.when(r1 + 1 < RPW)
        def _():
            gather(r1 + 1, rows0, sem0)

        drain(rows1, sem1)
        row_body(r1, rows1)
        return 0

    lax.fori_loop(0, RPW // 2, pair, 0)
    pltpu.sync_copy(outb, out_hbm.at[pl.ds(base * D, RPW * D)])


@jax.jit
def kernel(h, w, Wv0, bv0, Wo0, bo0, Wv1, bv1, Wo1, bo1, Wg, bg):
    BT = 256
    BR = 512

    t16, rinv16, gmax = pl.pallas_call(
        _thresh_body,
        grid=(N // BT,),
        in_specs=[pl.BlockSpec((BT, N), lambda i: (i, 0))],
        out_specs=[
            pl.BlockSpec((BT, 16), lambda i: (i, 0)),
            pl.BlockSpec((BT, 16), lambda i: (i, 0)),
            pl.BlockSpec((BT, 128), lambda i: (i, 0)),
        ],
        out_shape=[
            jax.ShapeDtypeStruct((N, 16), jnp.float32),
            jax.ShapeDtypeStruct((N, 16), jnp.float32),
            jax.ShapeDtypeStruct((N, 128), jnp.float32),
        ],
    )(w)

    idx64, wn64 = _sc_compact(
        w, t16.reshape(-1), rinv16.reshape(-1), gmax.reshape(-1)
    )

    vproj = pl.pallas_call(
        _vproj_body,
        grid=(N // BR,),
        in_specs=[
            pl.BlockSpec((BR, D), lambda i: (i, 0)),
            pl.BlockSpec((D, D), lambda i: (0, 0)),
            pl.BlockSpec((1, D), lambda i: (0, 0)),
        ],
        out_specs=pl.BlockSpec((BR, D), lambda i: (i, 0)),
        out_shape=jax.ShapeDtypeStruct((N, D), jnp.float32),
    )

    gate = pl.pallas_call(
        _gate_body,
        grid=(N // BR,),
        in_specs=[
            pl.BlockSpec((BR, D), lambda i: (i, 0)),
            pl.BlockSpec((BR, D), lambda i: (i, 0)),
            pl.BlockSpec((D, D), lambda i: (0, 0)),
            pl.BlockSpec((1, D), lambda i: (0, 0)),
            pl.BlockSpec((D, 1), lambda i: (0, 0)),
            pl.BlockSpec((1, 1), lambda i: (0, 0)),
        ],
        out_specs=pl.BlockSpec((BR, D), lambda i: (i, 0)),
        out_shape=jax.ShapeDtypeStruct((N, D), jnp.float32),
    )

    bg2 = bg.reshape(1, 1)
    for (Wv, bv, Wo, bo) in ((Wv0, bv0, Wo0, bo0), (Wv1, bv1, Wo1, bo1)):
        V = vproj(h, Wv, bv.reshape(1, D))
        msg = _sc_aggregate(V, idx64, wn64).reshape(N, D)
        h = gate(h, msg, Wo, bo.reshape(1, D), Wg, bg2)
    return h
